# baseline (speedup 1.0000x reference)
import functools

import jax
import jax.numpy as jnp
from jax import lax
from jax.experimental import pallas as pl
from jax.experimental.pallas import tpu as pltpu
from jax.experimental.pallas import tpu_sc as plsc

_CHUNK = 32
_NBUF = 4


@functools.lru_cache(maxsize=None)
def _build(B, V, D):
    info = plsc.get_sparse_core_info()
    NC, NS, L = info.num_cores, info.num_subcores, info.num_lanes
    NW = NC * NS
    assert B % (NW * _CHUNK * _NBUF) == 0
    b_per_w = B // NW
    n_chunks = b_per_w // _CHUNK
    n_groups = n_chunks // _NBUF
    mesh = plsc.VectorSubcoreMesh(core_axis_name="c", subcore_axis_name="s")

    @functools.partial(
        pl.kernel,
        mesh=mesh,
        out_type=jax.ShapeDtypeStruct((B, D), jnp.float32),
        scratch_types=[
            pltpu.VMEM((n_chunks, _CHUNK), jnp.int32),
        ] + [pltpu.VMEM((_CHUNK, D), jnp.float32) for _ in range(_NBUF)]
          + [pltpu.SemaphoreType.DMA for _ in range(2 * _NBUF)],
    )
    def gather_kernel(idx_hbm, table_hbm, out_hbm, idx_v, *bufs_sems):
        rows = bufs_sems[:_NBUF]
        g_sems = bufs_sems[_NBUF:2 * _NBUF]
        o_sems = bufs_sems[2 * _NBUF:]
        wid = lax.axis_index("s") * NC + lax.axis_index("c")
        base = wid * b_per_w

        pltpu.sync_copy(idx_hbm.at[wid], idx_v)
        for c in range(n_chunks):
            row = idx_v.at[c]
            for i in range(_CHUNK // L):
                sl = pl.ds(i * L, L)
                row[sl] = jnp.clip(row[sl], 0, V - 1)

        def start_gather(c, b):
            return pltpu.async_copy(table_hbm.at[idx_v.at[c]], rows[b],
                                    g_sems[b])

        def start_out(c, b, g):
            off = base + (g * _NBUF + b) * _CHUNK
            return pltpu.async_copy(rows[b], out_hbm.at[pl.ds(off, _CHUNK)],
                                    o_sems[b])

        def wait_gather(b):
            pltpu.make_async_copy(table_hbm.at[idx_v.at[0]], rows[b],
                                  g_sems[b]).wait()

        def wait_out(b):
            pltpu.make_async_copy(rows[b], out_hbm.at[pl.ds(base, _CHUNK)],
                                  o_sems[b]).wait()

        for b in range(_NBUF):
            start_gather(b, b)

        def group(g, _):
            for b in range(_NBUF):
                wait_gather(b)
                start_out(0, b, g)
            @pl.when(g + 1 < n_groups)
            def _():
                for b in range(_NBUF):
                    wait_out(b)
                    start_gather((g + 1) * _NBUF + b, b)
            return 0

        lax.fori_loop(0, n_groups, group, 0)
        for b in range(_NBUF):
            wait_out(b)

    return gather_kernel


def kernel(input, table):
    B = input.shape[0] * input.shape[1]
    NW = 32
    idx = jnp.reshape(input, (NW, B // (NW * _CHUNK), _CHUNK)).astype(jnp.int32)
    out = _build(B, table.shape[0], table.shape[1])(idx, table)
    return jnp.reshape(out, input.shape + (table.shape[1],))
